# baseline (speedup 1.0000x reference)
import functools

import jax
import jax.numpy as jnp
from jax import lax
from jax.experimental import pallas as pl
from jax.experimental.pallas import tpu as pltpu
from jax.experimental.pallas import tpu_sc as plsc

_B, _T, _D, _E, _TOP_K = 4, 4096, 4096, 64, 8
_TM = 512
_NC, _NS, _L = 2, 16, 16
_NW = _NC * _NS


def _mm_kernel(x_ref, w_ref, out_ref):
    out_ref[...] = jax.lax.dot_general(
        x_ref[...], w_ref[...],
        dimension_numbers=(((1,), (1,)), ((), ())),
        preferred_element_type=jnp.float32,
    )


def _gate_logits(xf, W):
    m = xf.shape[0]
    return pl.pallas_call(
        _mm_kernel,
        grid=(m // _TM,),
        in_specs=[
            pl.BlockSpec((_TM, _D), lambda i: (i, 0)),
            pl.BlockSpec((_E, _D), lambda i: (0, 0)),
        ],
        out_specs=pl.BlockSpec((_TM, _E), lambda i: (i, 0)),
        out_shape=jax.ShapeDtypeStruct((m, _E), jnp.float32),
        compiler_params=pltpu.CompilerParams(
            dimension_semantics=("parallel",),
        ),
    )(xf, W)


def _make_topk_sc(m):
    rows_per_w = m // _NW
    n_groups = rows_per_w // _L
    mesh = plsc.VectorSubcoreMesh(
        core_axis_name="c", subcore_axis_name="s",
        num_cores=_NC, num_subcores=_NS,
    )

    @functools.partial(
        pl.kernel,
        out_type=[
            jax.ShapeDtypeStruct((m * _TOP_K,), jnp.int32),
            jax.ShapeDtypeStruct((m * _TOP_K,), jnp.float32),
        ],
        mesh=mesh,
        scratch_types=[
            pltpu.VMEM((rows_per_w * _E,), jnp.float32),
            pltpu.VMEM((rows_per_w * _TOP_K,), jnp.int32),
            pltpu.VMEM((rows_per_w * _TOP_K,), jnp.float32),
        ],
        compiler_params=pltpu.CompilerParams(needs_layout_passes=False),
    )
    def topk_kernel(lg_hbm, oi_hbm, ow_hbm, buf, oi_v, ow_v):
        wid = lax.axis_index("s") * _NC + lax.axis_index("c")
        base = wid * rows_per_w
        pltpu.sync_copy(lg_hbm.at[pl.ds(base * _E, rows_per_w * _E)], buf)
        iota = lax.broadcasted_iota(jnp.int32, (_L,), 0)

        def group(g, gcarry):
            rows = g * _L + iota

            def ebody(e, carry):
                vs, ix = carry
                ev = jnp.broadcast_to(e, (_L,))
                xv = plsc.load_gather(buf, [rows * _E + ev])
                c = [xv > vs[j] for j in range(_TOP_K)]
                nv, ni = [], []
                for j in range(_TOP_K):
                    iv = jnp.where(c[j], xv, vs[j])
                    ii = jnp.where(c[j], ev, ix[j])
                    if j:
                        iv = jnp.where(c[j - 1], vs[j - 1], iv)
                        ii = jnp.where(c[j - 1], ix[j - 1], ii)
                    nv.append(iv)
                    ni.append(ii)
                return tuple(nv), tuple(ni)

            neg = jnp.full((_L,), -jnp.inf, jnp.float32)
            zero = jnp.zeros((_L,), jnp.int32)
            vs, ix = lax.fori_loop(
                0, _E, ebody, ((neg,) * _TOP_K, (zero,) * _TOP_K))

            exps = [jnp.exp(vs[j] - vs[0]) for j in range(_TOP_K)]
            s = exps[0]
            for j in range(1, _TOP_K):
                s = s + exps[j]
            r = 1.0 / s
            for j in range(_TOP_K):
                col = rows * _TOP_K + j
                plsc.store_scatter(oi_v, [col], ix[j])
                plsc.store_scatter(ow_v, [col], exps[j] * r)
            return gcarry

        lax.fori_loop(0, n_groups, group, 0)
        pltpu.sync_copy(oi_v, oi_hbm.at[pl.ds(base * _TOP_K, rows_per_w * _TOP_K)])
        pltpu.sync_copy(ow_v, ow_hbm.at[pl.ds(base * _TOP_K, rows_per_w * _TOP_K)])

    return topk_kernel


def kernel(x, W):
    m = _B * _T
    xf = x.reshape(m, _D)
    logits = _gate_logits(xf, W)
    idx, wts = _make_topk_sc(m)(logits.reshape(m * _E))
    return idx.reshape(_B, _T, _TOP_K), wts.reshape(_B, _T, _TOP_K)
